# baseline (speedup 1.0000x reference)
import functools
import math

import jax
import jax.numpy as jnp
from jax import lax
from jax.experimental import pallas as pl
from jax.experimental.pallas import tpu as pltpu
from jax.experimental.pallas import tpu_sc as plsc

N_ROWS = 262144
B_OBS = 16384
IN_SZ = 16
HID = 64
PREP = 8

NC, NS, L = 2, 16, 16
NW = NC * NS
BPW = B_OBS // NW
RPW = N_ROWS // NW

_LOGC = math.log(math.sqrt(2.0 * math.pi))


def _sc_gather_body(h_hbm, p_hbm, idx_hbm, hob_hbm, pob_hbm,
                    idx_v, hbuf, pbuf, sem_h, sem_p):
    wid = lax.axis_index("s") * NC + lax.axis_index("c")
    base = wid * BPW
    pltpu.sync_copy(idx_hbm.at[pl.ds(wid * 4, 4)], idx_v)
    cps = []
    for j in range(4):
        cps.append(pltpu.async_copy(h_hbm.at[idx_v.at[j]],
                                    hbuf.at[pl.ds(j * 128, 128)], sem_h))
        cps.append(pltpu.async_copy(p_hbm.at[idx_v.at[j]],
                                    pbuf.at[pl.ds(j * 128, 128)], sem_p))
    for c in cps:
        c.wait()
    pltpu.sync_copy(hbuf, hob_hbm.at[pl.ds(base, BPW)])
    pltpu.sync_copy(pbuf, pob_hbm.at[pl.ds(base, BPW)])


_sc_gather = functools.partial(
    pl.kernel,
    _sc_gather_body,
    out_type=(jax.ShapeDtypeStruct((B_OBS, HID), jnp.float32),
              jax.ShapeDtypeStruct((B_OBS, 2 * IN_SZ), jnp.float32)),
    mesh=plsc.VectorSubcoreMesh(core_axis_name="c", subcore_axis_name="s"),
    compiler_params=pltpu.CompilerParams(use_tc_tiling_on_sc=False),
    scratch_types=[pltpu.VMEM((4, 128), jnp.int32),
                   pltpu.VMEM((BPW, HID), jnp.float32),
                   pltpu.VMEM((BPW, 2 * IN_SZ), jnp.float32),
                   pltpu.SemaphoreType.DMA,
                   pltpu.SemaphoreType.DMA],
)


KBUF = 8
CHR = 128
NCH = RPW // CHR


def _sc_scatter_body(h_hbm, hnew_hbm, idx_hbm, out_hbm,
                     idx_all, winner, s16, bsrc3, dst3, patch, cbuf,
                     sem_g, sem_s, *ring_sems):
    wid = lax.axis_index("s") * NC + lax.axis_index("c")
    base = wid * RPW
    sin = ring_sems[:KBUF]
    sout = ring_sems[KBUF:]

    din, dout = {}, {}
    for b in range(KBUF // 2):
        din[b] = pltpu.async_copy(
            h_hbm.at[pl.ds(base + b * CHR, CHR)], cbuf.at[b], sin[b])
    for ch in range(NCH):
        b = ch % KBUF
        chp = ch + KBUF // 2
        if chp < NCH:
            bp = chp % KBUF
            if chp >= KBUF:
                dout[chp - KBUF].wait()
            din[chp] = pltpu.async_copy(
                h_hbm.at[pl.ds(base + chp * CHR, CHR)], cbuf.at[bp], sin[bp])
        din[ch].wait()
        dout[ch] = pltpu.async_copy(
            cbuf.at[b], out_hbm.at[pl.ds(base + ch * CHR, CHR)], sout[b])
    for ch in range(NCH - KBUF, NCH):
        dout[ch].wait()

    pltpu.sync_copy(idx_hbm, idx_all)
    iota = lax.iota(jnp.int32, 16)
    neg1 = jnp.full((16,), -1, jnp.int32)

    def init_body(t, _):
        winner[pl.ds(t * 16, 16)] = neg1
        return 0
    lax.fori_loop(0, RPW // 16, init_body, 0)


    def scan_body(t, _):
        v = idx_all[pl.ds(t * 16, 16)]
        inr = (v >= base) & (v < base + RPW)
        local = v - base
        b = t * 16 + iota
        key = jnp.where(inr, (local << 14) | b, -1)
        ks = lax.sort(key)
        s16[...] = ks
        nxt = plsc.load_gather(s16, [jnp.minimum(iota + 1, 15)])
        keep = (ks >= 0) & (((ks >> 14) != (nxt >> 14)) | (iota == 15))
        kl = jnp.maximum(ks >> 14, 0)
        cur = plsc.load_gather(winner, [kl], mask=keep)
        plsc.store_scatter(winner, [kl], jnp.maximum(cur, ks), mask=keep)
        return 0
    lax.fori_loop(0, B_OBS // 16, scan_body, 0)

    def pre_body(t, carry):
        g, mk = carry
        wv = winner[pl.ds(t * 16, 16)]
        g = g + jnp.sum(jnp.where(wv >= 0, 1, 0))
        mk = jnp.maximum(mk, jnp.max(wv))
        return (g, mk)
    g_tot, maxk = lax.fori_loop(0, RPW // 16, pre_body,
                                (jnp.int32(0), jnp.int32(-1)))

    @pl.when(g_tot > 0)
    def _():
        fill_b = jnp.full((16,), maxk & (B_OBS - 1), jnp.int32)
        fill_d = jnp.full((16,), base + (maxk >> 14), jnp.int32)

        def fill_body(t, _):
            r = t // 8
            o = (t % 8) * 16
            bsrc3[r, 0, pl.ds(o, 16)] = fill_b
            dst3[r, 0, pl.ds(o, 16)] = fill_d
            return 0
        lax.fori_loop(0, (RPW // 128) * 8, fill_body, 0)

        zero16 = jnp.zeros((16,), jnp.int32)

        def rank_body(t, run):
            wv = winner[pl.ds(t * 16, 16)]
            valid = wv >= 0
            vi = jnp.where(valid, 1, 0)
            cum = plsc.cumsum(vi)
            pos = run + cum - 1
            bval = wv & (B_OBS - 1)
            dval = base + (wv >> 14)
            plsc.store_scatter(bsrc3, [pos >> 7, zero16, pos & 127],
                               bval, mask=valid)
            plsc.store_scatter(dst3, [pos >> 7, zero16, pos & 127],
                               dval, mask=valid)
            return run + jnp.sum(vi)
        lax.fori_loop(0, RPW // 16, rank_body, jnp.int32(0))

        nstr = (g_tot + 127) // 128

        def str_body(j, _):
            gcp = pltpu.async_copy(hnew_hbm.at[bsrc3.at[j, 0]], patch, sem_g)
            gcp.wait()
            scp = pltpu.async_copy(patch, out_hbm.at[dst3.at[j, 0]], sem_s)
            scp.wait()
            return 0
        lax.fori_loop(0, nstr, str_body, 0)


_sc_scatter = functools.partial(
    pl.kernel,
    _sc_scatter_body,
    out_type=jax.ShapeDtypeStruct((N_ROWS, HID), jnp.float32),
    mesh=plsc.VectorSubcoreMesh(core_axis_name="c", subcore_axis_name="s"),
    compiler_params=pltpu.CompilerParams(use_tc_tiling_on_sc=False,
                                         needs_layout_passes=False),
    scratch_types=([pltpu.VMEM((B_OBS,), jnp.int32),
                    pltpu.VMEM((RPW,), jnp.int32),
                    pltpu.VMEM((16,), jnp.int32),
                    pltpu.VMEM((RPW // 128, 1, 128), jnp.int32),
                    pltpu.VMEM((RPW // 128, 1, 128), jnp.int32),
                    pltpu.VMEM((128, HID), jnp.float32),
                    pltpu.VMEM((KBUF, CHR, HID), jnp.float32),
                    pltpu.SemaphoreType.DMA,
                    pltpu.SemaphoreType.DMA]
                   + [pltpu.SemaphoreType.DMA] * (2 * KBUF)),
)


def _prep_body(x_ref, m_ref, p_ref, wbig_ref, bbig_ref, losses_ref, c_ref):
    x = x_ref[...]
    m = m_ref[...]
    pob = p_ref[...]
    mean = pob[:, :IN_SZ]
    logvar = pob[:, IN_SZ:]
    err = (x - mean) * jnp.exp(-0.5 * logvar)
    losses_ref[...] = 0.5 * ((err * err + logvar + 2.0 * _LOGC) * m)
    stack = jnp.concatenate([x, mean, logvar, err], axis=1)
    c = jnp.dot(stack, wbig_ref[...], preferred_element_type=jnp.float32)
    c = jnp.maximum(c + bbig_ref[...], 0.0)
    r = m.shape[0]
    m_rep = jnp.broadcast_to(m[:, :, None], (r, IN_SZ, PREP)).reshape(r, IN_SZ * PREP)
    c_ref[...] = c * m_rep


def _gru_body(xin_ref, hob_ref, gk_ref, grk_ref, gib_ref, grb_ref, hnew_ref):
    x = xin_ref[...]
    h0 = hob_ref[...]
    mx = jnp.dot(x, gk_ref[...], preferred_element_type=jnp.float32) + gib_ref[...]
    mi = jnp.dot(h0, grk_ref[...], preferred_element_type=jnp.float32) + grb_ref[...]
    z = jax.nn.sigmoid(mx[:, :HID] + mi[:, :HID])
    r = jax.nn.sigmoid(mx[:, HID:2 * HID] + mi[:, HID:2 * HID])
    hh = jnp.tanh(mx[:, 2 * HID:] + r * mi[:, 2 * HID:])
    hnew_ref[...] = z * h0 + (1.0 - z) * hh


def kernel(h, p, X_obs, M_obs, i_obs, w_prep, bias_prep, gru_kernel,
           gru_rec_kernel, gru_input_bias, gru_rec_bias):
    eye = jnp.eye(IN_SZ, dtype=jnp.float32)
    W_big = (jnp.transpose(w_prep, (1, 0, 2))[:, :, None, :]
             * eye[None, :, :, None]).reshape(4 * IN_SZ, IN_SZ * PREP)
    bias_big = bias_prep.reshape(1, IN_SZ * PREP)

    idx2d = i_obs.reshape(B_OBS // 128, 128)
    h_obs, p_obs = _sc_gather()(h, p, idx2d)

    R = 2048
    grid = (B_OBS // R,)
    losses, c = pl.pallas_call(
        _prep_body,
        grid=grid,
        in_specs=[pl.BlockSpec((R, IN_SZ), lambda i: (i, 0)),
                  pl.BlockSpec((R, IN_SZ), lambda i: (i, 0)),
                  pl.BlockSpec((R, 2 * IN_SZ), lambda i: (i, 0)),
                  pl.BlockSpec((4 * IN_SZ, IN_SZ * PREP), lambda i: (0, 0)),
                  pl.BlockSpec((1, IN_SZ * PREP), lambda i: (0, 0))],
        out_specs=[pl.BlockSpec((R, IN_SZ), lambda i: (i, 0)),
                   pl.BlockSpec((R, IN_SZ * PREP), lambda i: (i, 0))],
        out_shape=[jax.ShapeDtypeStruct((B_OBS, IN_SZ), jnp.float32),
                   jax.ShapeDtypeStruct((B_OBS, IN_SZ * PREP), jnp.float32)],
    )(X_obs, M_obs, p_obs, W_big, bias_big)

    gru_in = (c.reshape(B_OBS * IN_SZ, PREP)
                .transpose(1, 0)
                .reshape(B_OBS, IN_SZ * PREP))

    h_new = pl.pallas_call(
        _gru_body,
        grid=grid,
        in_specs=[pl.BlockSpec((R, IN_SZ * PREP), lambda i: (i, 0)),
                  pl.BlockSpec((R, HID), lambda i: (i, 0)),
                  pl.BlockSpec((IN_SZ * PREP, 3 * HID), lambda i: (0, 0)),
                  pl.BlockSpec((HID, 3 * HID), lambda i: (0, 0)),
                  pl.BlockSpec((1, 3 * HID), lambda i: (0, 0)),
                  pl.BlockSpec((1, 3 * HID), lambda i: (0, 0))],
        out_specs=pl.BlockSpec((R, HID), lambda i: (i, 0)),
        out_shape=jax.ShapeDtypeStruct((B_OBS, HID), jnp.float32),
    )(gru_in, h_obs, gru_kernel, gru_rec_kernel,
      gru_input_bias.reshape(1, 3 * HID), gru_rec_bias.reshape(1, 3 * HID))

    h_out = _sc_scatter()(h, h_new, i_obs)
    return (h_out, losses)


# speedup vs baseline: 1.0704x; 1.0704x over previous
import functools
import math

import jax
import jax.numpy as jnp
from jax import lax
from jax.experimental import pallas as pl
from jax.experimental.pallas import tpu as pltpu
from jax.experimental.pallas import tpu_sc as plsc

N_ROWS = 262144
B_OBS = 16384
IN_SZ = 16
HID = 64
PREP = 8

NC, NS, L = 2, 16, 16
NW = NC * NS
BPW = B_OBS // NW
RPW = N_ROWS // NW

_LOGC = math.log(math.sqrt(2.0 * math.pi))


def _sc_gather_body(h_hbm, p_hbm, idx_hbm, hob_hbm, pob_hbm,
                    idx_v, hbuf, pbuf, sem):
    wid = lax.axis_index("s") * NC + lax.axis_index("c")
    base = wid * BPW
    pltpu.sync_copy(idx_hbm.at[pl.ds(wid * 4, 4)], idx_v)
    for half in range(2):
        groups = {}
        for g in range(16):
            gg = half * 16 + g
            r, cc = divmod(gg, 8)
            wv = idx_v[r, pl.ds(cc * 16, 16)]
            ds = []
            for j in range(16):
                s = wv[j]
                row = g * 16 + j
                ds.append(pltpu.async_copy(h_hbm.at[s], hbuf.at[row], sem))
                ds.append(pltpu.async_copy(p_hbm.at[s], pbuf.at[row], sem))
            groups[g] = ds
            if g >= 1:
                for d in groups[g - 1]:
                    d.wait()
        for d in groups[15]:
            d.wait()
        pltpu.sync_copy(hbuf, hob_hbm.at[pl.ds(base + half * 256, 256)])
        pltpu.sync_copy(pbuf, pob_hbm.at[pl.ds(base + half * 256, 256)])


_sc_gather = functools.partial(
    pl.kernel,
    _sc_gather_body,
    out_type=(jax.ShapeDtypeStruct((B_OBS, HID), jnp.float32),
              jax.ShapeDtypeStruct((B_OBS, 2 * IN_SZ), jnp.float32)),
    mesh=plsc.VectorSubcoreMesh(core_axis_name="c", subcore_axis_name="s"),
    compiler_params=pltpu.CompilerParams(needs_layout_passes=False),
    scratch_types=[pltpu.VMEM((4, 128), jnp.int32),
                   pltpu.VMEM((256, HID), jnp.float32),
                   pltpu.VMEM((256, 2 * IN_SZ), jnp.float32),
                   pltpu.SemaphoreType.DMA],
)


KBUF = 8
CHR = 128
NCH = RPW // CHR


def _sc_scatter_body(h_hbm, hnew_hbm, idx_hbm, out_hbm,
                     idx_all, winner, s16, bsrc3, dst3, patch, cbuf,
                     sem_g, sem_s, *ring_sems):
    wid = lax.axis_index("s") * NC + lax.axis_index("c")
    base = wid * RPW
    sin = ring_sems[:KBUF]
    sout = ring_sems[KBUF:]

    din, dout = {}, {}
    for b in range(KBUF // 2):
        din[b] = pltpu.async_copy(
            h_hbm.at[pl.ds(base + b * CHR, CHR)], cbuf.at[b], sin[b])
    for ch in range(NCH):
        b = ch % KBUF
        chp = ch + KBUF // 2
        if chp < NCH:
            bp = chp % KBUF
            if chp >= KBUF:
                dout[chp - KBUF].wait()
            din[chp] = pltpu.async_copy(
                h_hbm.at[pl.ds(base + chp * CHR, CHR)], cbuf.at[bp], sin[bp])
        din[ch].wait()
        dout[ch] = pltpu.async_copy(
            cbuf.at[b], out_hbm.at[pl.ds(base + ch * CHR, CHR)], sout[b])
    for ch in range(NCH - KBUF, NCH):
        dout[ch].wait()

    pltpu.sync_copy(idx_hbm, idx_all)
    iota = lax.iota(jnp.int32, 16)
    neg1 = jnp.full((16,), -1, jnp.int32)

    def init_body(t, _):
        winner[pl.ds(t * 16, 16)] = neg1
        return 0
    lax.fori_loop(0, RPW // 16, init_body, 0)


    def scan_body(t, _):
        v = idx_all[pl.ds(t * 16, 16)]
        inr = (v >= base) & (v < base + RPW)
        local = v - base
        b = t * 16 + iota
        key = jnp.where(inr, (local << 14) | b, -1)
        ks = lax.sort(key)
        s16[...] = ks
        nxt = plsc.load_gather(s16, [jnp.minimum(iota + 1, 15)])
        keep = (ks >= 0) & (((ks >> 14) != (nxt >> 14)) | (iota == 15))
        kl = jnp.maximum(ks >> 14, 0)
        cur = plsc.load_gather(winner, [kl], mask=keep)
        plsc.store_scatter(winner, [kl], jnp.maximum(cur, ks), mask=keep)
        return 0
    lax.fori_loop(0, B_OBS // 16, scan_body, 0)

    def pre_body(t, carry):
        g, mk = carry
        wv = winner[pl.ds(t * 16, 16)]
        g = g + jnp.sum(jnp.where(wv >= 0, 1, 0))
        mk = jnp.maximum(mk, jnp.max(wv))
        return (g, mk)
    g_tot, maxk = lax.fori_loop(0, RPW // 16, pre_body,
                                (jnp.int32(0), jnp.int32(-1)))

    @pl.when(g_tot > 0)
    def _():
        fill_b = jnp.full((16,), maxk & (B_OBS - 1), jnp.int32)
        fill_d = jnp.full((16,), base + (maxk >> 14), jnp.int32)

        def fill_body(t, _):
            r = t // 8
            o = (t % 8) * 16
            bsrc3[r, 0, pl.ds(o, 16)] = fill_b
            dst3[r, 0, pl.ds(o, 16)] = fill_d
            return 0
        lax.fori_loop(0, (RPW // 128) * 8, fill_body, 0)

        zero16 = jnp.zeros((16,), jnp.int32)

        def rank_body(t, run):
            wv = winner[pl.ds(t * 16, 16)]
            valid = wv >= 0
            vi = jnp.where(valid, 1, 0)
            cum = plsc.cumsum(vi)
            pos = run + cum - 1
            bval = wv & (B_OBS - 1)
            dval = base + (wv >> 14)
            plsc.store_scatter(bsrc3, [pos >> 7, zero16, pos & 127],
                               bval, mask=valid)
            plsc.store_scatter(dst3, [pos >> 7, zero16, pos & 127],
                               dval, mask=valid)
            return run + jnp.sum(vi)
        lax.fori_loop(0, RPW // 16, rank_body, jnp.int32(0))

        nstr = (g_tot + 127) // 128

        def str_body(j, _):
            gcp = pltpu.async_copy(hnew_hbm.at[bsrc3.at[j, 0]], patch, sem_g)
            gcp.wait()
            scp = pltpu.async_copy(patch, out_hbm.at[dst3.at[j, 0]], sem_s)
            scp.wait()
            return 0
        lax.fori_loop(0, nstr, str_body, 0)


_sc_scatter = functools.partial(
    pl.kernel,
    _sc_scatter_body,
    out_type=jax.ShapeDtypeStruct((N_ROWS, HID), jnp.float32),
    mesh=plsc.VectorSubcoreMesh(core_axis_name="c", subcore_axis_name="s"),
    compiler_params=pltpu.CompilerParams(use_tc_tiling_on_sc=False,
                                         needs_layout_passes=False),
    scratch_types=([pltpu.VMEM((B_OBS,), jnp.int32),
                    pltpu.VMEM((RPW,), jnp.int32),
                    pltpu.VMEM((16,), jnp.int32),
                    pltpu.VMEM((RPW // 128, 1, 128), jnp.int32),
                    pltpu.VMEM((RPW // 128, 1, 128), jnp.int32),
                    pltpu.VMEM((128, HID), jnp.float32),
                    pltpu.VMEM((KBUF, CHR, HID), jnp.float32),
                    pltpu.SemaphoreType.DMA,
                    pltpu.SemaphoreType.DMA]
                   + [pltpu.SemaphoreType.DMA] * (2 * KBUF)),
)


def _prep_body(x_ref, m_ref, p_ref, wbig_ref, bbig_ref, losses_ref, c_ref):
    x = x_ref[...]
    m = m_ref[...]
    pob = p_ref[...]
    mean = pob[:, :IN_SZ]
    logvar = pob[:, IN_SZ:]
    err = (x - mean) * jnp.exp(-0.5 * logvar)
    losses_ref[...] = 0.5 * ((err * err + logvar + 2.0 * _LOGC) * m)
    stack = jnp.concatenate([x, mean, logvar, err], axis=1)
    c = jnp.dot(stack, wbig_ref[...], preferred_element_type=jnp.float32)
    c = jnp.maximum(c + bbig_ref[...], 0.0)
    r = m.shape[0]
    m_rep = jnp.broadcast_to(m[:, :, None], (r, IN_SZ, PREP)).reshape(r, IN_SZ * PREP)
    c_ref[...] = c * m_rep


def _gru_body(xin_ref, hob_ref, gk_ref, grk_ref, gib_ref, grb_ref, hnew_ref):
    x = xin_ref[...]
    h0 = hob_ref[...]
    mx = jnp.dot(x, gk_ref[...], preferred_element_type=jnp.float32) + gib_ref[...]
    mi = jnp.dot(h0, grk_ref[...], preferred_element_type=jnp.float32) + grb_ref[...]
    z = jax.nn.sigmoid(mx[:, :HID] + mi[:, :HID])
    r = jax.nn.sigmoid(mx[:, HID:2 * HID] + mi[:, HID:2 * HID])
    hh = jnp.tanh(mx[:, 2 * HID:] + r * mi[:, 2 * HID:])
    hnew_ref[...] = z * h0 + (1.0 - z) * hh


def kernel(h, p, X_obs, M_obs, i_obs, w_prep, bias_prep, gru_kernel,
           gru_rec_kernel, gru_input_bias, gru_rec_bias):
    eye = jnp.eye(IN_SZ, dtype=jnp.float32)
    W_big = (jnp.transpose(w_prep, (1, 0, 2))[:, :, None, :]
             * eye[None, :, :, None]).reshape(4 * IN_SZ, IN_SZ * PREP)
    bias_big = bias_prep.reshape(1, IN_SZ * PREP)

    idx2d = i_obs.reshape(B_OBS // 128, 128)
    h_obs, p_obs = _sc_gather()(h, p, idx2d)

    R = 2048
    grid = (B_OBS // R,)
    losses, c = pl.pallas_call(
        _prep_body,
        grid=grid,
        in_specs=[pl.BlockSpec((R, IN_SZ), lambda i: (i, 0)),
                  pl.BlockSpec((R, IN_SZ), lambda i: (i, 0)),
                  pl.BlockSpec((R, 2 * IN_SZ), lambda i: (i, 0)),
                  pl.BlockSpec((4 * IN_SZ, IN_SZ * PREP), lambda i: (0, 0)),
                  pl.BlockSpec((1, IN_SZ * PREP), lambda i: (0, 0))],
        out_specs=[pl.BlockSpec((R, IN_SZ), lambda i: (i, 0)),
                   pl.BlockSpec((R, IN_SZ * PREP), lambda i: (i, 0))],
        out_shape=[jax.ShapeDtypeStruct((B_OBS, IN_SZ), jnp.float32),
                   jax.ShapeDtypeStruct((B_OBS, IN_SZ * PREP), jnp.float32)],
    )(X_obs, M_obs, p_obs, W_big, bias_big)

    gru_in = (c.reshape(B_OBS, IN_SZ, PREP)
                .transpose(2, 0, 1)
                .reshape(B_OBS, IN_SZ * PREP))

    h_new = pl.pallas_call(
        _gru_body,
        grid=grid,
        in_specs=[pl.BlockSpec((R, IN_SZ * PREP), lambda i: (i, 0)),
                  pl.BlockSpec((R, HID), lambda i: (i, 0)),
                  pl.BlockSpec((IN_SZ * PREP, 3 * HID), lambda i: (0, 0)),
                  pl.BlockSpec((HID, 3 * HID), lambda i: (0, 0)),
                  pl.BlockSpec((1, 3 * HID), lambda i: (0, 0)),
                  pl.BlockSpec((1, 3 * HID), lambda i: (0, 0))],
        out_specs=pl.BlockSpec((R, HID), lambda i: (i, 0)),
        out_shape=jax.ShapeDtypeStruct((B_OBS, HID), jnp.float32),
    )(gru_in, h_obs, gru_kernel, gru_rec_kernel,
      gru_input_bias.reshape(1, 3 * HID), gru_rec_bias.reshape(1, 3 * HID))

    h_out = _sc_scatter()(h, h_new, i_obs)
    return (h_out, losses)


# speedup vs baseline: 1.1009x; 1.0285x over previous
import functools
import math

import jax
import jax.numpy as jnp
from jax import lax
from jax.experimental import pallas as pl
from jax.experimental.pallas import tpu as pltpu
from jax.experimental.pallas import tpu_sc as plsc

N_ROWS = 262144
B_OBS = 16384
IN_SZ = 16
HID = 64
PREP = 8

NC, NS, L = 2, 16, 16
NW = NC * NS
BPW = B_OBS // NW
RPW = N_ROWS // NW

_LOGC = math.log(math.sqrt(2.0 * math.pi))


def _sc_gather_body(h_hbm, p_hbm, idx_hbm, hob_hbm, pob_hbm,
                    idx_v, hbuf, pbuf, sem_h, sem_p):
    wid = lax.axis_index("s") * NC + lax.axis_index("c")
    base = wid * BPW
    pltpu.sync_copy(idx_hbm.at[pl.ds(wid * 4, 4)], idx_v)
    cps = []
    for j in range(4):
        cps.append(pltpu.async_copy(h_hbm.at[idx_v.at[j]],
                                    hbuf.at[pl.ds(j * 128, 128)], sem_h))
        cps.append(pltpu.async_copy(p_hbm.at[idx_v.at[j]],
                                    pbuf.at[pl.ds(j * 128, 128)], sem_p))
    for c in cps:
        c.wait()
    pltpu.sync_copy(hbuf, hob_hbm.at[pl.ds(base, BPW)])
    pltpu.sync_copy(pbuf, pob_hbm.at[pl.ds(base, BPW)])


_sc_gather = functools.partial(
    pl.kernel,
    _sc_gather_body,
    out_type=(jax.ShapeDtypeStruct((B_OBS, HID), jnp.float32),
              jax.ShapeDtypeStruct((B_OBS, 2 * IN_SZ), jnp.float32)),
    mesh=plsc.VectorSubcoreMesh(core_axis_name="c", subcore_axis_name="s"),
    compiler_params=pltpu.CompilerParams(use_tc_tiling_on_sc=False),
    scratch_types=[pltpu.VMEM((4, 128), jnp.int32),
                   pltpu.VMEM((BPW, HID), jnp.float32),
                   pltpu.VMEM((BPW, 2 * IN_SZ), jnp.float32),
                   pltpu.SemaphoreType.DMA,
                   pltpu.SemaphoreType.DMA],
)


KBUF = 4
CHR = 256
NCH = RPW // CHR


def _sc_scatter_body(h_hbm, hnew_hbm, idx_hbm, out_hbm,
                     idx_all, winner, s16, bsrc3, dst3, patch, cbuf,
                     sem_g, sem_s, *ring_sems):
    wid = lax.axis_index("s") * NC + lax.axis_index("c")
    base = wid * RPW
    sin = ring_sems[:KBUF]
    sout = ring_sems[KBUF:]

    din, dout = {}, {}
    for b in range(KBUF // 2):
        din[b] = pltpu.async_copy(
            h_hbm.at[pl.ds(base + b * CHR, CHR)], cbuf.at[b], sin[b])
    for ch in range(NCH):
        b = ch % KBUF
        chp = ch + KBUF // 2
        if chp < NCH:
            bp = chp % KBUF
            if chp >= KBUF:
                dout[chp - KBUF].wait()
            din[chp] = pltpu.async_copy(
                h_hbm.at[pl.ds(base + chp * CHR, CHR)], cbuf.at[bp], sin[bp])
        din[ch].wait()
        dout[ch] = pltpu.async_copy(
            cbuf.at[b], out_hbm.at[pl.ds(base + ch * CHR, CHR)], sout[b])
    for ch in range(NCH - KBUF, NCH):
        dout[ch].wait()

    pltpu.sync_copy(idx_hbm, idx_all)
    iota = lax.iota(jnp.int32, 16)
    neg1 = jnp.full((16,), -1, jnp.int32)

    def init_body(t, _):
        winner[pl.ds(t * 16, 16)] = neg1
        return 0
    lax.fori_loop(0, RPW // 16, init_body, 0)


    def scan_body(t, _):
        v = idx_all[pl.ds(t * 16, 16)]
        inr = (v >= base) & (v < base + RPW)
        local = v - base
        b = t * 16 + iota
        key = jnp.where(inr, (local << 14) | b, -1)
        ks = lax.sort(key)
        s16[...] = ks
        nxt = plsc.load_gather(s16, [jnp.minimum(iota + 1, 15)])
        keep = (ks >= 0) & (((ks >> 14) != (nxt >> 14)) | (iota == 15))
        kl = jnp.maximum(ks >> 14, 0)
        cur = plsc.load_gather(winner, [kl], mask=keep)
        plsc.store_scatter(winner, [kl], jnp.maximum(cur, ks), mask=keep)
        return 0
    lax.fori_loop(0, B_OBS // 16, scan_body, 0)

    def pre_body(t, carry):
        g, mk = carry
        wv = winner[pl.ds(t * 16, 16)]
        g = g + jnp.sum(jnp.where(wv >= 0, 1, 0))
        mk = jnp.maximum(mk, jnp.max(wv))
        return (g, mk)
    g_tot, maxk = lax.fori_loop(0, RPW // 16, pre_body,
                                (jnp.int32(0), jnp.int32(-1)))

    @pl.when(g_tot > 0)
    def _():
        fill_b = jnp.full((16,), maxk & (B_OBS - 1), jnp.int32)
        fill_d = jnp.full((16,), base + (maxk >> 14), jnp.int32)

        def fill_body(t, _):
            r = t // 8
            o = (t % 8) * 16
            bsrc3[r, 0, pl.ds(o, 16)] = fill_b
            dst3[r, 0, pl.ds(o, 16)] = fill_d
            return 0
        lax.fori_loop(0, (RPW // 128) * 8, fill_body, 0)

        zero16 = jnp.zeros((16,), jnp.int32)

        def rank_body(t, run):
            wv = winner[pl.ds(t * 16, 16)]
            valid = wv >= 0
            vi = jnp.where(valid, 1, 0)
            cum = plsc.cumsum(vi)
            pos = run + cum - 1
            bval = wv & (B_OBS - 1)
            dval = base + (wv >> 14)
            plsc.store_scatter(bsrc3, [pos >> 7, zero16, pos & 127],
                               bval, mask=valid)
            plsc.store_scatter(dst3, [pos >> 7, zero16, pos & 127],
                               dval, mask=valid)
            return run + jnp.sum(vi)
        lax.fori_loop(0, RPW // 16, rank_body, jnp.int32(0))

        nstr = (g_tot + 127) // 128

        def str_body(j, _):
            gcp = pltpu.async_copy(hnew_hbm.at[bsrc3.at[j, 0]], patch, sem_g)
            gcp.wait()
            scp = pltpu.async_copy(patch, out_hbm.at[dst3.at[j, 0]], sem_s)
            scp.wait()
            return 0
        lax.fori_loop(0, nstr, str_body, 0)


_sc_scatter = functools.partial(
    pl.kernel,
    _sc_scatter_body,
    out_type=jax.ShapeDtypeStruct((N_ROWS, HID), jnp.float32),
    mesh=plsc.VectorSubcoreMesh(core_axis_name="c", subcore_axis_name="s"),
    compiler_params=pltpu.CompilerParams(use_tc_tiling_on_sc=False,
                                         needs_layout_passes=False),
    scratch_types=([pltpu.VMEM((B_OBS,), jnp.int32),
                    pltpu.VMEM((RPW,), jnp.int32),
                    pltpu.VMEM((16,), jnp.int32),
                    pltpu.VMEM((RPW // 128, 1, 128), jnp.int32),
                    pltpu.VMEM((RPW // 128, 1, 128), jnp.int32),
                    pltpu.VMEM((128, HID), jnp.float32),
                    pltpu.VMEM((KBUF, CHR, HID), jnp.float32),
                    pltpu.SemaphoreType.DMA,
                    pltpu.SemaphoreType.DMA]
                   + [pltpu.SemaphoreType.DMA] * (2 * KBUF)),
)


def _prep_body(x_ref, m_ref, p_ref, wbig_ref, bbig_ref, losses_ref, c_ref):
    x = x_ref[...]
    m = m_ref[...]
    pob = p_ref[...]
    mean = pob[:, :IN_SZ]
    logvar = pob[:, IN_SZ:]
    err = (x - mean) * jnp.exp(-0.5 * logvar)
    losses_ref[...] = 0.5 * ((err * err + logvar + 2.0 * _LOGC) * m)
    stack = jnp.concatenate([x, mean, logvar, err], axis=1)
    c = jnp.dot(stack, wbig_ref[...], preferred_element_type=jnp.float32)
    c = jnp.maximum(c + bbig_ref[...], 0.0)
    r = m.shape[0]
    m_rep = jnp.broadcast_to(m[:, :, None], (r, IN_SZ, PREP)).reshape(r, IN_SZ * PREP)
    c_ref[...] = c * m_rep


def _gru_body(xin_ref, hob_ref, gk_ref, grk_ref, gib_ref, grb_ref, hnew_ref):
    x = xin_ref[...]
    h0 = hob_ref[...]
    mx = jnp.dot(x, gk_ref[...], preferred_element_type=jnp.float32) + gib_ref[...]
    mi = jnp.dot(h0, grk_ref[...], preferred_element_type=jnp.float32) + grb_ref[...]
    z = jax.nn.sigmoid(mx[:, :HID] + mi[:, :HID])
    r = jax.nn.sigmoid(mx[:, HID:2 * HID] + mi[:, HID:2 * HID])
    hh = jnp.tanh(mx[:, 2 * HID:] + r * mi[:, 2 * HID:])
    hnew_ref[...] = z * h0 + (1.0 - z) * hh


def kernel(h, p, X_obs, M_obs, i_obs, w_prep, bias_prep, gru_kernel,
           gru_rec_kernel, gru_input_bias, gru_rec_bias):
    eye = jnp.eye(IN_SZ, dtype=jnp.float32)
    W_big = (jnp.transpose(w_prep, (1, 0, 2))[:, :, None, :]
             * eye[None, :, :, None]).reshape(4 * IN_SZ, IN_SZ * PREP)
    bias_big = bias_prep.reshape(1, IN_SZ * PREP)

    idx2d = i_obs.reshape(B_OBS // 128, 128)
    h_obs, p_obs = _sc_gather()(h, p, idx2d)

    R = 2048
    grid = (B_OBS // R,)
    losses, c = pl.pallas_call(
        _prep_body,
        grid=grid,
        in_specs=[pl.BlockSpec((R, IN_SZ), lambda i: (i, 0)),
                  pl.BlockSpec((R, IN_SZ), lambda i: (i, 0)),
                  pl.BlockSpec((R, 2 * IN_SZ), lambda i: (i, 0)),
                  pl.BlockSpec((4 * IN_SZ, IN_SZ * PREP), lambda i: (0, 0)),
                  pl.BlockSpec((1, IN_SZ * PREP), lambda i: (0, 0))],
        out_specs=[pl.BlockSpec((R, IN_SZ), lambda i: (i, 0)),
                   pl.BlockSpec((R, IN_SZ * PREP), lambda i: (i, 0))],
        out_shape=[jax.ShapeDtypeStruct((B_OBS, IN_SZ), jnp.float32),
                   jax.ShapeDtypeStruct((B_OBS, IN_SZ * PREP), jnp.float32)],
    )(X_obs, M_obs, p_obs, W_big, bias_big)

    gru_in = (c.reshape(B_OBS, IN_SZ, PREP)
                .transpose(2, 0, 1)
                .reshape(B_OBS, IN_SZ * PREP))

    h_new = pl.pallas_call(
        _gru_body,
        grid=grid,
        in_specs=[pl.BlockSpec((R, IN_SZ * PREP), lambda i: (i, 0)),
                  pl.BlockSpec((R, HID), lambda i: (i, 0)),
                  pl.BlockSpec((IN_SZ * PREP, 3 * HID), lambda i: (0, 0)),
                  pl.BlockSpec((HID, 3 * HID), lambda i: (0, 0)),
                  pl.BlockSpec((1, 3 * HID), lambda i: (0, 0)),
                  pl.BlockSpec((1, 3 * HID), lambda i: (0, 0))],
        out_specs=pl.BlockSpec((R, HID), lambda i: (i, 0)),
        out_shape=jax.ShapeDtypeStruct((B_OBS, HID), jnp.float32),
    )(gru_in, h_obs, gru_kernel, gru_rec_kernel,
      gru_input_bias.reshape(1, 3 * HID), gru_rec_bias.reshape(1, 3 * HID))

    h_out = _sc_scatter()(h, h_new, i_obs)
    return (h_out, losses)
